# baseline (speedup 1.0000x reference)
import jax
import jax.numpy as jnp
from jax import lax
from jax.experimental import pallas as pl
from jax.experimental.pallas import tpu as pltpu
from jax.experimental.pallas import tpu_sc as plsc

B = 16384
D = 16
F = 26
DNUM = 13
OUT_W = DNUM + F * D

NC = 2
NS = 16
NW = NC * NS
BPW = B // NW
CH = 128
NCH = BPW // CH
IDX_W = F * NCH * CH


def _body(*refs):
    numeric = refs[0]
    cats = refs[1]
    tables = refs[2:2 + F]
    out = refs[2 + F]
    idx_v, embbuf, numbuf, rowbuf, sem = refs[3 + F:]

    wid = lax.axis_index("s") * NC + lax.axis_index("c")
    base = wid * BPW

    pltpu.sync_copy(cats.at[wid], idx_v)
    ones = jnp.ones((16,), jnp.int32)

    @pl.loop(0, IDX_W // 16)
    def shift(i):
        idx_v[pl.ds(i * 16, 16)] = idx_v[pl.ds(i * 16, 16)] + ones

    @pl.loop(0, NCH)
    def chunk(c):
        rowbase = base + c * CH
        coff = pl.multiple_of(c * CH, CH)
        pltpu.sync_copy(numeric.at[pl.ds(rowbase, CH), :], numbuf)
        copies = []
        for t in range(F):
            ioff = pl.multiple_of(t * NCH * CH + coff, CH)
            copies.append(pltpu.async_copy(
                tables[t].at[idx_v.at[pl.ds(ioff, CH)]],
                embbuf.at[pl.ds(t * CH, CH), :],
                sem,
            ))
        for cp in copies:
            cp.wait()

        @pl.loop(0, CH)
        def asm(r):
            rowbuf[r, pl.ds(0, 16)] = numbuf[r, :]
            for t in range(F):
                rowbuf[r, pl.ds(DNUM + D * t, D)] = embbuf[t * CH + r, :]

        pltpu.sync_copy(rowbuf, out.at[pl.ds(rowbase, CH), :])


@jax.jit
def _run(numeric, cats, tables):
    kern = pl.kernel(
        _body,
        out_type=jax.ShapeDtypeStruct((B, OUT_W), jnp.float32),
        mesh=plsc.VectorSubcoreMesh(
            core_axis_name="c", subcore_axis_name="s",
            num_cores=NC, num_subcores=NS,
        ),
        scratch_types=[
            pltpu.VMEM((IDX_W,), jnp.int32),
            pltpu.VMEM((F * CH, D), jnp.float32),
            pltpu.VMEM((CH, 16), jnp.float32),
            pltpu.VMEM((CH, OUT_W), jnp.float32),
            pltpu.SemaphoreType.DMA,
        ],
        compiler_params=pltpu.CompilerParams(use_tc_tiling_on_sc=False),
    )
    return kern(numeric, cats, *tables)


def kernel(numeric, cat_0, cat_1, cat_2, cat_3, cat_4, cat_5, cat_6, cat_7, cat_8, cat_9, cat_10, cat_11, cat_12, cat_13, cat_14, cat_15, cat_16, cat_17, cat_18, cat_19, cat_20, cat_21, cat_22, cat_23, cat_24, cat_25, W_0, W_1, W_2, W_3, W_4, W_5, W_6, W_7, W_8, W_9, W_10, W_11, W_12, W_13, W_14, W_15, W_16, W_17, W_18, W_19, W_20, W_21, W_22, W_23, W_24, W_25):
    cats = (cat_0, cat_1, cat_2, cat_3, cat_4, cat_5, cat_6, cat_7, cat_8,
            cat_9, cat_10, cat_11, cat_12, cat_13, cat_14, cat_15, cat_16,
            cat_17, cat_18, cat_19, cat_20, cat_21, cat_22, cat_23, cat_24,
            cat_25)
    tables = (W_0, W_1, W_2, W_3, W_4, W_5, W_6, W_7, W_8, W_9, W_10, W_11,
              W_12, W_13, W_14, W_15, W_16, W_17, W_18, W_19, W_20, W_21,
              W_22, W_23, W_24, W_25)
    cats_w = (
        jnp.stack(cats, 0)
        .reshape(F, NW, NCH * CH)
        .transpose(1, 0, 2)
        .reshape(NW, IDX_W)
    )
    numeric16 = jnp.pad(numeric, ((0, 0), (0, 16 - DNUM)))
    return _run(numeric16, cats_w, tables)
